# Initial kernel scaffold; baseline (speedup 1.0000x reference)
import jax
import jax.numpy as jnp
from jax.experimental import pallas as pl


def kernel(x, source, target, edge_type, base_weights, bases):
    raise NotImplementedError("write your pallas kernel here")


# speedup vs baseline: 4.6700x; 4.6700x over previous
import functools

import jax
import jax.numpy as jnp
from jax import lax
from jax.experimental import pallas as pl
from jax.experimental.pallas import tpu as pltpu
from jax.experimental.pallas import tpu_sc as plsc

N_NODES = 10000
N_EDGES = 320000
N_REL = 16
N_BASES = 4
DIM = 128

NC = 2
NS = 16
NW = NC * NS
EPW = N_EDGES // NW
K = 80
NCHUNK = EPW // K


def _v_body(bw_ref, bases_ref, x_ref, v_ref):
    r = pl.program_id(1)
    rows = lax.broadcasted_iota(jnp.int32, (N_REL, N_BASES), 0)
    bvec = jnp.sum(jnp.where(rows == r, bw_ref[...], 0.0), axis=0)
    w = bvec[0] * bases_ref[0]
    for b in range(1, N_BASES):
        w = w + bvec[b] * bases_ref[b]
    v_ref[0] = jnp.dot(x_ref[...], w, preferred_element_type=jnp.float32)


def _compute_v(x, base_weights, bases, n_tile=1000):
    nt = N_NODES // n_tile
    return pl.pallas_call(
        _v_body,
        grid=(nt, N_REL),
        in_specs=[
            pl.BlockSpec((N_REL, N_BASES), lambda n, r: (0, 0)),
            pl.BlockSpec((N_BASES, DIM, DIM), lambda n, r: (0, 0, 0)),
            pl.BlockSpec((n_tile, DIM), lambda n, r: (n, 0)),
        ],
        out_specs=pl.BlockSpec((1, n_tile, DIM), lambda n, r: (r, n, 0)),
        out_shape=jax.ShapeDtypeStruct((N_REL, N_NODES, DIM), jnp.float32),
    )(base_weights, bases, x)


def _gidx_body(src_ref, et_ref, gidx_ref):
    gidx_ref[...] = et_ref[...] * N_NODES + src_ref[...]


def _compute_gidx(source, edge_type):
    src2 = source.reshape(N_EDGES // 128, 128)
    et2 = edge_type.reshape(N_EDGES // 128, 128)
    out = pl.pallas_call(
        _gidx_body,
        out_shape=jax.ShapeDtypeStruct((N_EDGES // 128, 128), jnp.int32),
    )(src2, et2)
    return out.reshape(N_EDGES)


def _sc_edge_body(gidx_hbm, tgt_hbm, v_hbm, out_hbm,
                  gidx_v, tgt_v, rows_v, zbuf, acc, sem):
    c = lax.axis_index("c")
    s = lax.axis_index("s")
    wid = c * NS + s

    zero = jnp.zeros((16,), jnp.float32)
    for i in range(16):
        for j in range(DIM // 16):
            zbuf[i, pl.ds(j * 16, 16)] = zero

    nblk_total = N_NODES // 16
    my_nblk = (nblk_total - s + NS - 1) // NS

    def zero_blk(i, _):
        blk = s + i * NS
        pltpu.sync_copy(zbuf, acc.at[pl.ds(blk * 16, 16), :])
        return 0

    lax.fori_loop(0, my_nblk, zero_blk, 0)
    plsc.subcore_barrier()

    base = wid * EPW

    def chunk(i, _):
        off = base + i * K
        pltpu.sync_copy(gidx_hbm.at[pl.ds(off, K)], gidx_v)
        pltpu.sync_copy(tgt_hbm.at[pl.ds(off, K)], tgt_v)
        pltpu.async_copy(v_hbm.at[gidx_v], rows_v, sem).wait()
        pltpu.sync_copy(rows_v, acc.at[tgt_v], add=True)
        return 0

    lax.fori_loop(0, NCHUNK, chunk, 0)
    plsc.subcore_barrier()

    def out_blk(i, _):
        blk = s + i * NS
        pltpu.sync_copy(acc.at[pl.ds(blk * 16, 16), :],
                        out_hbm.at[c, pl.ds(blk * 16, 16), :])
        return 0

    lax.fori_loop(0, my_nblk, out_blk, 0)


def _sc_edge(gidx, target, v):
    mesh = plsc.VectorSubcoreMesh(core_axis_name="c", subcore_axis_name="s")
    kern = pl.kernel(
        _sc_edge_body,
        out_type=jax.ShapeDtypeStruct((NC, N_NODES, DIM), jnp.float32),
        mesh=mesh,
        scratch_types=[
            pltpu.VMEM((K,), jnp.int32),
            pltpu.VMEM((K,), jnp.int32),
            pltpu.VMEM((K, DIM), jnp.float32),
            pltpu.VMEM((16, DIM), jnp.float32),
            pltpu.VMEM_SHARED((N_NODES, DIM), jnp.float32),
            pltpu.SemaphoreType.DMA,
        ],
    )
    return kern(gidx, target, v.reshape(N_REL * N_NODES, DIM))


def _sum_body(p_ref, o_ref):
    o_ref[...] = p_ref[0] + p_ref[1]


def _sum_partials(partial, n_tile=1000):
    nt = N_NODES // n_tile
    return pl.pallas_call(
        _sum_body,
        grid=(nt,),
        in_specs=[pl.BlockSpec((NC, n_tile, DIM), lambda n: (0, n, 0))],
        out_specs=pl.BlockSpec((n_tile, DIM), lambda n: (n, 0)),
        out_shape=jax.ShapeDtypeStruct((N_NODES, DIM), jnp.float32),
    )(partial)


def kernel(x, source, target, edge_type, base_weights, bases):
    v = _compute_v(x, base_weights, bases)
    gidx = _compute_gidx(source, edge_type)
    partial = _sc_edge(gidx, target, v)
    return _sum_partials(partial)
